# baseline (speedup 1.0000x reference)
import functools

import jax
import jax.numpy as jnp
from jax import lax
from jax.experimental import pallas as pl
from jax.experimental.pallas import tpu as pltpu
from jax.experimental.pallas import tpu_sc as plsc

N_ROWS = 320000
D = 128
N_SEG = 10000
NC = 2
NS = 16
PIECE = 128
N_PIECES = N_ROWS // PIECE
NBUF = 5
HALF = N_SEG // 2
TRASH = HALF
ACC_ROWS = HALF + 8
ZROWS = 312
ZTAIL = HALF - ZROWS * NS
NSS = 4
SUB = PIECE // NSS


def _sc_segment_sum(data, ids):
    mesh = plsc.VectorSubcoreMesh(
        core_axis_name="c", subcore_axis_name="s", num_cores=NC, num_subcores=NS
    )

    @functools.partial(
        pl.kernel,
        out_type=jax.ShapeDtypeStruct((N_SEG, D), jnp.float32),
        mesh=mesh,
        scratch_types=[
            pltpu.VMEM((NBUF, PIECE, D), jnp.float32),
            pltpu.VMEM((NBUF, NSS, PIECE // NSS), jnp.int32),
            pltpu.VMEM((16, 16), jnp.int32),
            pltpu.VMEM_SHARED((ACC_ROWS, D), jnp.float32),
        ]
        + [pltpu.SemaphoreType.DMA] * (3 * NBUF + 2),
    )
    def seg_sum_kernel(data_hbm, ids_hbm, out_hbm, db, ib, sb, acc, *sems):
        sem_g = sems[:NBUF]
        sem_i = sems[NBUF : 2 * NBUF]
        sem_s = sems[2 * NBUF : 3 * NBUF]
        sem_z = sems[3 * NBUF]
        sem_p = sems[3 * NBUF + 1]
        c = lax.axis_index("c")
        s = lax.axis_index("s")

        def row0_of(j):
            p = jnp.where(c == 0, s + NS * j, (N_PIECES - 1) - s - NS * j)
            return pl.multiple_of(p * PIECE, PIECE)

        def start_gather(j, b):
            r0 = row0_of(j)
            half_p = PIECE // 2
            pltpu.async_copy(
                data_hbm.at[pl.ds(r0, half_p)], db.at[b, pl.ds(0, half_p)], sem_g[b]
            )
            pltpu.async_copy(
                data_hbm.at[pl.ds(r0 + half_p, half_p)],
                db.at[b, pl.ds(half_p, half_p)],
                sem_g[b],
            )
            for m in range(NSS):
                pltpu.async_copy(
                    ids_hbm.at[pl.ds(r0 + m * SUB, SUB)], ib.at[b, m], sem_i[b]
                )

        def wait_gather(b):
            for _h in range(2):
                pltpu.make_async_copy(
                    data_hbm.at[pl.ds(0, PIECE // 2)],
                    db.at[b, pl.ds(0, PIECE // 2)],
                    sem_g[b],
                ).wait()
            for m in range(NSS):
                pltpu.make_async_copy(
                    ids_hbm.at[pl.ds(0, SUB)], ib.at[b, m], sem_i[b]
                ).wait()

        def start_scatter(b):
            for m in range(NSS):
                pltpu.async_copy(
                    db.at[b, pl.ds(m * SUB, SUB)],
                    acc.at[ib.at[b, m]],
                    sem_s[b],
                    add=True,
                )

        def wait_scatter(b):
            for m in range(NSS):
                pltpu.make_async_copy(
                    db.at[0, pl.ds(0, SUB)], acc.at[ib.at[0, 0]], sem_s[b]
                ).wait()

        for j in range(NBUF - 1):
            start_gather(j, j)

        zeros16 = jnp.zeros((16,), jnp.float32)
        zb = NBUF - 1

        def zrow(r, carry):
            def zlane(l, cc):
                db[zb, r, pl.ds(l * 16, 16)] = zeros16
                return cc
            return lax.fori_loop(0, D // 16, zlane, carry)

        lax.fori_loop(0, PIECE, zrow, 0)
        zbase = s * ZROWS
        zcopies = [(zbase, PIECE), (zbase + PIECE, PIECE), (zbase + 256, ZROWS - 256)]
        for off, n in zcopies:
            pltpu.async_copy(db.at[zb, pl.ds(0, n)], acc.at[pl.ds(off, n)], sem_z)

        @pl.when(s == NS - 1)
        def _():
            pltpu.async_copy(
                db.at[zb, pl.ds(0, ZTAIL)],
                acc.at[pl.ds(NS * ZROWS, ZTAIL)],
                sem_z,
            )

        def probe_round(lo, step):
            for i in range(16):
                q = jnp.minimum(lo + i * step, N_PIECES - 1)
                pltpu.async_copy(
                    ids_hbm.at[pl.ds(pl.multiple_of(q * PIECE, PIECE), 16)],
                    sb.at[i],
                    sem_p,
                )
            for i in range(16):
                pltpu.make_async_copy(
                    ids_hbm.at[pl.ds(0, 16)], sb.at[i], sem_p
                ).wait()
            nf = jnp.int32(0)
            for i in range(16):
                v = sb[i, pl.ds(0, 16)]
                nf = nf + jnp.where(v[0] < HALF, 1, 0).astype(jnp.int32)
            return nf

        lo = jnp.int32(0)
        nf1 = probe_round(lo, 157)
        pred0 = nf1 == 0
        lo = lo + (jnp.maximum(nf1, 1) - 1) * 157
        nf2 = probe_round(lo, 10)
        lo = lo + (jnp.maximum(nf2, 1) - 1) * 10
        nf3 = probe_round(lo, 1)
        qb = jnp.minimum(lo + nf3, N_PIECES)
        qb = jnp.where(pred0, 0, qb)
        sp = jnp.maximum(qb - 1, 0)

        for off, n in zcopies:
            pltpu.make_async_copy(
                db.at[zb, pl.ds(0, n)], acc.at[pl.ds(off, n)], sem_z
            ).wait()

        @pl.when(s == NS - 1)
        def _():
            pltpu.make_async_copy(
                db.at[zb, pl.ds(0, ZTAIL)], acc.at[pl.ds(NS * ZROWS, ZTAIL)], sem_z
            ).wait()

        plsc.subcore_barrier()

        n_sc = jnp.where(c == 0, sp + 1, N_PIECES - sp)
        n_j = (n_sc - s + NS - 1) // NS

        def remap(b):
            half = jnp.full((16,), HALF, jnp.int32)
            trash = jnp.full((16,), TRASH, jnp.int32)
            for m in range(NSS):
                for g in range(SUB // 16):
                    t = ib[b, m, pl.ds(g * 16, 16)] - c * HALF
                    t = jnp.where((t < 0) | (t >= half), trash, t)
                    ib[b, m, pl.ds(g * 16, 16)] = t

        def quad(jj, carry):
            for k in range(NBUF):
                j = jj * NBUF + k
                b3 = (k + NBUF - 1) % NBUF

                @pl.when(j < n_j)
                def _(j=j, k=k, b3=b3):
                    @pl.when(j + (NBUF - 1) < n_j)
                    def _():
                        @pl.when(j >= 1)
                        def _():
                            wait_scatter(b3)

                        start_gather(j + (NBUF - 1), b3)

                    wait_gather(k)
                    remap(k)
                    start_scatter(k)
            return carry

        lax.fori_loop(0, (n_j + NBUF - 1) // NBUF, quad, 0)

        for j in range(NBUF - 1):
            @pl.when(n_j <= j)
            def _(j=j):
                wait_gather(j)

        for b in range(NBUF):
            @pl.when(n_j >= b + 1)
            def _(b=b):
                wait_scatter(b)

        plsc.subcore_barrier()

        pltpu.sync_copy(
            acc.at[pl.ds(zbase, ZROWS)],
            out_hbm.at[pl.ds(c * HALF + zbase, ZROWS)],
        )

        @pl.when(s == NS - 1)
        def _():
            pltpu.sync_copy(
                acc.at[pl.ds(NS * ZROWS, ZTAIL)],
                out_hbm.at[pl.ds(c * HALF + NS * ZROWS, ZTAIL)],
            )

    return seg_sum_kernel(data, ids)


@jax.jit
def kernel(data, segment_ids):
    return _sc_segment_sum(data, segment_ids)
